# Initial kernel scaffold; baseline (speedup 1.0000x reference)
import functools

import jax
import jax.numpy as jnp
from jax import lax
from jax.experimental import pallas as pl
from jax.experimental.pallas import tpu as pltpu
from jax.experimental.pallas import tpu_sc as plsc

NC = 2
NS = 16
NW = NC * NS
L = 16


@functools.partial(jax.jit, static_argnames=("B", "K", "D"))
def _sc_scores(pu, pv, ng, embed_u, embed_v, *, B, K, D):
    RPW = B // NW
    NBLK = RPW // L
    NSUB = RPW // 128
    mesh = plsc.VectorSubcoreMesh(core_axis_name="c", subcore_axis_name="s")

    @functools.partial(
        pl.kernel,
        out_type=(jax.ShapeDtypeStruct((B,), jnp.float32),
                  jax.ShapeDtypeStruct((K, B), jnp.float32)),
        mesh=mesh,
        scratch_types=[
            pltpu.VMEM((NSUB, 128), jnp.int32),
            pltpu.VMEM((NSUB, 128), jnp.int32),
            pltpu.VMEM((RPW * K // 128, 128), jnp.int32),
            pltpu.VMEM((NSUB, 128), jnp.int32),
            pltpu.VMEM((RPW, D), jnp.float32),
            pltpu.VMEM((RPW, D), jnp.float32),
            pltpu.VMEM((RPW,), jnp.float32),
            pltpu.SemaphoreType.DMA,
        ],
    )
    def sc_kernel(pu_hbm, pv_hbm, ng_hbm, eu_hbm, ev_hbm,
                  out_pos, out_neg,
                  pu_idx, pv_idx, ng_idx, col_idx, u_rows, w_rows,
                  scores, sem):
        wid = lax.axis_index("s") * NC + lax.axis_index("c")
        base = wid * RPW

        pltpu.sync_copy(pu_hbm.at[pl.ds(wid * NSUB, NSUB)], pu_idx)
        pltpu.sync_copy(pv_hbm.at[pl.ds(wid * NSUB, NSUB)], pv_idx)
        pltpu.sync_copy(ng_hbm.at[pl.ds(wid * (RPW * K // 128),
                                        RPW * K // 128)], ng_idx)

        descs = []
        for i in range(NSUB):
            descs.append(pltpu.async_copy(
                eu_hbm.at[pu_idx.at[i]],
                u_rows.at[pl.ds(i * 128, 128)], sem))
            descs.append(pltpu.async_copy(
                ev_hbm.at[pv_idx.at[i]],
                w_rows.at[pl.ds(i * 128, 128)], sem))
        for dsc in descs:
            dsc.wait()

        def dot_blocks(other_ref):
            def blk(j, carry):
                rows = j * L + lax.iota(jnp.int32, L)

                def dstep(dd, acc):
                    cols = jnp.full((L,), dd, jnp.int32)
                    uc = plsc.load_gather(u_rows, [rows, cols])
                    oc = plsc.load_gather(other_ref, [rows, cols])
                    return acc + uc * oc

                acc = lax.fori_loop(0, D, dstep,
                                    jnp.zeros((L,), jnp.float32))
                scores[pl.ds(j * L, L)] = acc
                return carry

            lax.fori_loop(0, NBLK, blk, 0)

        dot_blocks(w_rows)
        pltpu.sync_copy(scores, out_pos.at[pl.ds(base, RPW)])

        def k_body(kk, carry):
            for j in range(NBLK):
                rows = j * L + lax.iota(jnp.int32, L)
                flat = rows * K + kk
                g = plsc.load_gather(ng_idx, [flat // 128, flat % 128])
                col_idx[j // 8, pl.ds((j % 8) * L, L)] = g
            nd = []
            for i in range(NSUB):
                nd.append(pltpu.async_copy(
                    ev_hbm.at[col_idx.at[i]],
                    w_rows.at[pl.ds(i * 128, 128)], sem))
            for dsc in nd:
                dsc.wait()
            dot_blocks(w_rows)
            pltpu.sync_copy(scores, out_neg.at[kk, pl.ds(base, RPW)])
            return carry

        lax.fori_loop(0, K, k_body, 0)

    return sc_kernel(pu, pv, ng, embed_u, embed_v)


@functools.partial(jax.jit, static_argnames=("B",))
def _tc_loss(pos_score2d, neg_score, *, B):
    def body(p_ref, n_ref, o_ref):
        def log_sig(x):
            return jnp.minimum(x, 0.0) - jnp.log1p(jnp.exp(-jnp.abs(x)))

        tot = jnp.sum(log_sig(p_ref[...])) + jnp.sum(log_sig(-n_ref[...]))
        o_ref[0, 0] = -tot / B

    out = pl.pallas_call(
        body,
        out_shape=jax.ShapeDtypeStruct((1, 1), jnp.float32),
        out_specs=pl.BlockSpec(memory_space=pltpu.SMEM),
    )(pos_score2d, neg_score)
    return out[0, 0]


def kernel(pos_u, pos_v, neg_v, embed_u, embed_v):
    B, K = neg_v.shape
    D = embed_u.shape[1]
    pu = pos_u.astype(jnp.int32).reshape(B // 128, 128)
    pv = pos_v.astype(jnp.int32).reshape(B // 128, 128)
    ng = neg_v.astype(jnp.int32).reshape(B * K // 128, 128)
    pos_s, neg_s = _sc_scores(pu, pv, ng, embed_u, embed_v, B=B, K=K, D=D)
    return _tc_loss(pos_s.reshape(128, B // 128), neg_s, B=B)


# speedup vs baseline: 3.4557x; 3.4557x over previous
import functools

import jax
import jax.numpy as jnp
from jax import lax
from jax.experimental import pallas as pl
from jax.experimental.pallas import tpu as pltpu
from jax.experimental.pallas import tpu_sc as plsc

NC = 2
NS = 16
NW = NC * NS
L = 16


@functools.partial(jax.jit, static_argnames=("B", "K", "D"))
def _sc_scores(pu, pv, ng, embed_u, embed_v, *, B, K, D):
    RPW = B // NW
    NBLK = RPW // L
    NSUB = RPW // 128
    mesh = plsc.VectorSubcoreMesh(core_axis_name="c", subcore_axis_name="s")

    @functools.partial(
        pl.kernel,
        out_type=(jax.ShapeDtypeStruct((B,), jnp.float32),
                  jax.ShapeDtypeStruct((K, B), jnp.float32)),
        mesh=mesh,
        compiler_params=pltpu.CompilerParams(needs_layout_passes=False,
                                             use_tc_tiling_on_sc=False),
        scratch_types=[
            pltpu.VMEM((NSUB, 128), jnp.int32),
            pltpu.VMEM((NSUB, 128), jnp.int32),
            pltpu.VMEM((RPW * K // 128, 128), jnp.int32),
            pltpu.VMEM((NSUB, 128), jnp.int32),
            pltpu.VMEM((RPW, D), jnp.float32),
            pltpu.VMEM((RPW, D), jnp.float32),
            pltpu.VMEM((RPW,), jnp.float32),
            pltpu.SemaphoreType.DMA,
        ],
    )
    def sc_kernel(pu_hbm, pv_hbm, ng_hbm, eu_hbm, ev_hbm,
                  out_pos, out_neg,
                  pu_idx, pv_idx, ng_idx, col_idx, u_rows, w_rows,
                  scores, sem):
        wid = lax.axis_index("s") * NC + lax.axis_index("c")
        base = wid * RPW

        pltpu.sync_copy(pu_hbm.at[pl.ds(wid * NSUB, NSUB)], pu_idx)
        pltpu.sync_copy(pv_hbm.at[pl.ds(wid * NSUB, NSUB)], pv_idx)
        pltpu.sync_copy(ng_hbm.at[pl.ds(wid * (RPW * K // 128),
                                        RPW * K // 128)], ng_idx)

        descs = []
        for i in range(NSUB):
            descs.append(pltpu.async_copy(
                eu_hbm.at[pu_idx.at[i]],
                u_rows.at[pl.ds(i * 128, 128)], sem))
            descs.append(pltpu.async_copy(
                ev_hbm.at[pv_idx.at[i]],
                w_rows.at[pl.ds(i * 128, 128)], sem))
        for dsc in descs:
            dsc.wait()

        def dot_blocks(other_ref):
            def blk(j, carry):
                rows = j * L + lax.iota(jnp.int32, L)

                def dstep(dd, acc):
                    cols = jnp.full((L,), dd, jnp.int32)
                    uc = plsc.load_gather(u_rows, [rows, cols])
                    oc = plsc.load_gather(other_ref, [rows, cols])
                    return acc + uc * oc

                acc = lax.fori_loop(0, D, dstep,
                                    jnp.zeros((L,), jnp.float32))
                scores[pl.ds(j * L, L)] = acc
                return carry

            lax.fori_loop(0, NBLK, blk, 0)

        dot_blocks(w_rows)
        pltpu.sync_copy(scores, out_pos.at[pl.ds(base, RPW)])

        def k_body(kk, carry):
            for j in range(NBLK):
                rows = j * L + lax.iota(jnp.int32, L)
                flat = rows * K + kk
                g = plsc.load_gather(ng_idx, [flat // 128, flat % 128])
                col_idx[j // 8, pl.ds((j % 8) * L, L)] = g
            nd = []
            for i in range(NSUB):
                nd.append(pltpu.async_copy(
                    ev_hbm.at[col_idx.at[i]],
                    w_rows.at[pl.ds(i * 128, 128)], sem))
            for dsc in nd:
                dsc.wait()
            dot_blocks(w_rows)
            pltpu.sync_copy(scores, out_neg.at[kk, pl.ds(base, RPW)])
            return carry

        lax.fori_loop(0, K, k_body, 0)

    return sc_kernel(pu, pv, ng, embed_u, embed_v)


@functools.partial(jax.jit, static_argnames=("B",))
def _tc_loss(pos_score2d, neg_score, *, B):
    def body(p_ref, n_ref, o_ref):
        def log_sig(x):
            return jnp.minimum(x, 0.0) - jnp.log1p(jnp.exp(-jnp.abs(x)))

        tot = jnp.sum(log_sig(p_ref[...])) + jnp.sum(log_sig(-n_ref[...]))
        o_ref[0, 0] = -tot / B

    out = pl.pallas_call(
        body,
        out_shape=jax.ShapeDtypeStruct((1, 1), jnp.float32),
        out_specs=pl.BlockSpec(memory_space=pltpu.SMEM),
    )(pos_score2d, neg_score)
    return out[0, 0]


def kernel(pos_u, pos_v, neg_v, embed_u, embed_v):
    B, K = neg_v.shape
    D = embed_u.shape[1]
    pu = pos_u.astype(jnp.int32).reshape(B // 128, 128)
    pv = pos_v.astype(jnp.int32).reshape(B // 128, 128)
    ng = neg_v.astype(jnp.int32).reshape(B * K // 128, 128)
    pos_s, neg_s = _sc_scores(pu, pv, ng, embed_u, embed_v, B=B, K=K, D=D)
    return _tc_loss(pos_s.reshape(128, B // 128), neg_s, B=B)


# speedup vs baseline: 3.8299x; 1.1083x over previous
import functools

import jax
import jax.numpy as jnp
from jax import lax
from jax.experimental import pallas as pl
from jax.experimental.pallas import tpu as pltpu
from jax.experimental.pallas import tpu_sc as plsc

NC = 2
NS = 16
NW = NC * NS
L = 16


@functools.partial(jax.jit, static_argnames=("B", "K", "D"))
def _sc_scores(pu, pv, ng, embed_u, embed_v, *, B, K, D):
    RPW = B // NW
    NBLK = RPW // L
    NSUB = RPW // 128
    NGR = RPW * K // 128
    mesh = plsc.VectorSubcoreMesh(core_axis_name="c", subcore_axis_name="s")

    @functools.partial(
        pl.kernel,
        out_type=(jax.ShapeDtypeStruct((NW, RPW), jnp.float32),
                  jax.ShapeDtypeStruct((NW, RPW * K), jnp.float32)),
        mesh=mesh,
        compiler_params=pltpu.CompilerParams(needs_layout_passes=False,
                                             use_tc_tiling_on_sc=False),
        scratch_types=[
            pltpu.VMEM((NSUB, 128), jnp.int32),
            pltpu.VMEM((NSUB, 128), jnp.int32),
            pltpu.VMEM((NGR, 128), jnp.int32),
            pltpu.VMEM((RPW, D), jnp.float32),
            pltpu.VMEM((RPW, D), jnp.float32),
            pltpu.VMEM((RPW, D), jnp.float32),
            pltpu.VMEM((RPW,), jnp.float32),
            pltpu.VMEM((RPW * K,), jnp.float32),
            pltpu.SemaphoreType.DMA,
            pltpu.SemaphoreType.DMA,
            pltpu.SemaphoreType.DMA,
        ],
    )
    def sc_kernel(pu_hbm, pv_hbm, ng_hbm, eu_hbm, ev_hbm,
                  out_pos, out_neg,
                  pu_idx, pv_idx, ng_idx, u_rows, buf_a, buf_b,
                  s_pos, s_neg, sem_u, sem_a, sem_b):
        wid = lax.axis_index("s") * NC + lax.axis_index("c")

        pltpu.sync_copy(pu_hbm.at[pl.ds(wid * NSUB, NSUB)], pu_idx)
        pltpu.sync_copy(pv_hbm.at[pl.ds(wid * NSUB, NSUB)], pv_idx)
        uv_descs = []
        for i in range(NSUB):
            uv_descs.append(pltpu.async_copy(
                eu_hbm.at[pu_idx.at[i]],
                u_rows.at[pl.ds(i * 128, 128)], sem_u))
            uv_descs.append(pltpu.async_copy(
                ev_hbm.at[pv_idx.at[i]],
                buf_a.at[pl.ds(i * 128, 128)], sem_u))
        pltpu.sync_copy(ng_hbm.at[pl.ds(wid * NGR, NGR)], ng_idx)
        for i in range(NSUB):
            pltpu.async_copy(ev_hbm.at[ng_idx.at[i]],
                             buf_b.at[pl.ds(i * 128, 128)], sem_b)
        for dsc in uv_descs:
            dsc.wait()

        iota = lax.iota(jnp.int32, L)
        cols_c = [jnp.full((L,), dd, jnp.int32) for dd in range(D)]

        def dot_pass(other_ref, urows_of, score_base):
            def blk(j, carry):
                lrows = j * L + iota
                urows = urows_of(j)
                accs = [jnp.zeros((L,), jnp.float32) for _ in range(4)]
                for dd in range(D):
                    uc = plsc.load_gather(u_rows, [urows, cols_c[dd]])
                    oc = plsc.load_gather(other_ref, [lrows, cols_c[dd]])
                    accs[dd % 4] = accs[dd % 4] + uc * oc
                return carry, (accs[0] + accs[1]) + (accs[2] + accs[3])

            def blk_pos(j, carry):
                carry, acc = blk(j, carry)
                s_pos[pl.ds(j * L, L)] = acc
                return carry

            def blk_neg(j, carry):
                carry, acc = blk(j, carry)
                s_neg[pl.ds(score_base + j * L, L)] = acc
                return carry

            lax.fori_loop(0, NBLK, blk_pos if score_base is None else blk_neg,
                          0)

        dot_pass(buf_a, lambda j: j * L + iota, None)
        pltpu.sync_copy(s_pos, out_pos.at[wid])

        def wait_chunk(buf, row0):
            for s in range(NSUB):
                pltpu.make_async_copy(
                    ev_hbm.at[ng_idx.at[row0 + s]],
                    buf.at[pl.ds(s * 128, 128)],
                    sem_b if buf is buf_b else sem_a).wait()

        def issue_chunk(buf, row0):
            for s in range(NSUB):
                pltpu.async_copy(ev_hbm.at[ng_idx.at[row0 + s]],
                                 buf.at[pl.ds(s * 128, 128)],
                                 sem_b if buf is buf_b else sem_a)

        def neg_urows(c, j):
            return (c * RPW + j * L + iota) // K

        def pair(i, carry):
            c0 = 2 * i
            c1 = c0 + 1
            issue_chunk(buf_a, c1 * NSUB)
            wait_chunk(buf_b, c0 * NSUB)
            dot_pass(buf_b, functools.partial(neg_urows, c0), c0 * RPW)
            c2 = jnp.minimum(c0 + 2, K - 2)
            issue_chunk(buf_b, c2 * NSUB)
            wait_chunk(buf_a, c1 * NSUB)
            dot_pass(buf_a, functools.partial(neg_urows, c1), c1 * RPW)
            return carry

        lax.fori_loop(0, K // 2, pair, 0)
        wait_chunk(buf_b, (K - 2) * NSUB)
        pltpu.sync_copy(s_neg, out_neg.at[wid])

    return sc_kernel(pu, pv, ng, embed_u, embed_v)


@functools.partial(jax.jit, static_argnames=("B",))
def _tc_loss(pos_score, neg_score, *, B):
    def body(p_ref, n_ref, o_ref):
        def log_sig(x):
            return jnp.minimum(x, 0.0) - jnp.log1p(jnp.exp(-jnp.abs(x)))

        tot = jnp.sum(log_sig(p_ref[...])) + jnp.sum(log_sig(-n_ref[...]))
        o_ref[0, 0] = -tot / B

    out = pl.pallas_call(
        body,
        out_shape=jax.ShapeDtypeStruct((1, 1), jnp.float32),
        out_specs=pl.BlockSpec(memory_space=pltpu.SMEM),
    )(pos_score, neg_score)
    return out[0, 0]


def kernel(pos_u, pos_v, neg_v, embed_u, embed_v):
    B, K = neg_v.shape
    D = embed_u.shape[1]
    pu = pos_u.astype(jnp.int32).reshape(B // 128, 128)
    pv = pos_v.astype(jnp.int32).reshape(B // 128, 128)
    ng = neg_v.astype(jnp.int32).reshape(B * K // 128, 128)
    pos_s, neg_s = _sc_scores(pu, pv, ng, embed_u, embed_v, B=B, K=K, D=D)
    return _tc_loss(pos_s, neg_s, B=B)


# speedup vs baseline: 4.8752x; 1.2729x over previous
import functools

import jax
import jax.numpy as jnp
from jax import lax
from jax.experimental import pallas as pl
from jax.experimental.pallas import tpu as pltpu
from jax.experimental.pallas import tpu_sc as plsc

NC = 2
NS = 16
NW = NC * NS
L = 16


@functools.partial(jax.jit, static_argnames=("B", "K", "D"))
def _sc_scores(pu, pv, ng, embed_u, embed_v, *, B, K, D):
    RPW = B // NW
    NBLK = RPW // L
    NSUB = RPW // 128
    NGR = RPW * K // 128
    mesh = plsc.VectorSubcoreMesh(core_axis_name="c", subcore_axis_name="s")

    @functools.partial(
        pl.kernel,
        out_type=(jax.ShapeDtypeStruct((NW, RPW), jnp.float32),
                  jax.ShapeDtypeStruct((NW, RPW * K), jnp.float32)),
        mesh=mesh,
        compiler_params=pltpu.CompilerParams(needs_layout_passes=False,
                                             use_tc_tiling_on_sc=False),
        scratch_types=[
            pltpu.VMEM((NSUB, 128), jnp.int32),
            pltpu.VMEM((NSUB, 128), jnp.int32),
            pltpu.VMEM((NGR, 128), jnp.int32),
            pltpu.VMEM((RPW, D), jnp.float32),
            pltpu.VMEM((RPW, D), jnp.float32),
            pltpu.VMEM((RPW, D), jnp.float32),
            pltpu.VMEM((RPW,), jnp.float32),
            pltpu.VMEM((RPW * K,), jnp.float32),
            pltpu.SemaphoreType.DMA,
            pltpu.SemaphoreType.DMA,
            pltpu.SemaphoreType.DMA,
        ],
    )
    def sc_kernel(pu_hbm, pv_hbm, ng_hbm, eu_hbm, ev_hbm,
                  out_pos, out_neg,
                  pu_idx, pv_idx, ng_idx, u_rows, buf_a, buf_b,
                  s_pos, s_neg, sem_u, sem_a, sem_b):
        wid = lax.axis_index("s") * NC + lax.axis_index("c")

        pltpu.sync_copy(pu_hbm.at[pl.ds(wid * NSUB, NSUB)], pu_idx)
        pltpu.sync_copy(pv_hbm.at[pl.ds(wid * NSUB, NSUB)], pv_idx)
        uv_descs = []
        for i in range(NSUB):
            uv_descs.append(pltpu.async_copy(
                eu_hbm.at[pu_idx.at[i]],
                u_rows.at[pl.ds(i * 128, 128)], sem_u))
            uv_descs.append(pltpu.async_copy(
                ev_hbm.at[pv_idx.at[i]],
                buf_a.at[pl.ds(i * 128, 128)], sem_u))
        pltpu.sync_copy(ng_hbm.at[pl.ds(wid * NGR, NGR)], ng_idx)
        for i in range(NSUB):
            pltpu.async_copy(ev_hbm.at[ng_idx.at[i]],
                             buf_b.at[pl.ds(i * 128, 128)], sem_b)
        for dsc in uv_descs:
            dsc.wait()

        iota = lax.iota(jnp.int32, L)

        def dot_pass(other_ref, urows_of, score_base):
            def blk(j, carry):
                lrows = j * L + iota
                urows = urows_of(j)
                accs = [jnp.zeros((L,), jnp.float32) for _ in range(4)]
                for dd in range(D):
                    cols = (iota + dd) & (D - 1)
                    uc = plsc.load_gather(u_rows, [urows, cols])
                    oc = plsc.load_gather(other_ref, [lrows, cols])
                    accs[dd % 4] = accs[dd % 4] + uc * oc
                return carry, (accs[0] + accs[1]) + (accs[2] + accs[3])

            def blk_pos(j, carry):
                carry, acc = blk(j, carry)
                s_pos[pl.ds(j * L, L)] = acc
                return carry

            def blk_neg(j, carry):
                carry, acc = blk(j, carry)
                s_neg[pl.ds(score_base + j * L, L)] = acc
                return carry

            lax.fori_loop(0, NBLK, blk_pos if score_base is None else blk_neg,
                          0)

        dot_pass(buf_a, lambda j: j * L + iota, None)
        pltpu.sync_copy(s_pos, out_pos.at[wid])

        def wait_chunk(buf, row0):
            for s in range(NSUB):
                pltpu.make_async_copy(
                    ev_hbm.at[ng_idx.at[row0 + s]],
                    buf.at[pl.ds(s * 128, 128)],
                    sem_b if buf is buf_b else sem_a).wait()

        def issue_chunk(buf, row0):
            for s in range(NSUB):
                pltpu.async_copy(ev_hbm.at[ng_idx.at[row0 + s]],
                                 buf.at[pl.ds(s * 128, 128)],
                                 sem_b if buf is buf_b else sem_a)

        def neg_urows(c, j):
            return (c * RPW + j * L + iota) // K

        def pair(i, carry):
            c0 = 2 * i
            c1 = c0 + 1
            issue_chunk(buf_a, c1 * NSUB)
            wait_chunk(buf_b, c0 * NSUB)
            dot_pass(buf_b, functools.partial(neg_urows, c0), c0 * RPW)
            c2 = jnp.minimum(c0 + 2, K - 2)
            issue_chunk(buf_b, c2 * NSUB)
            wait_chunk(buf_a, c1 * NSUB)
            dot_pass(buf_a, functools.partial(neg_urows, c1), c1 * RPW)
            return carry

        lax.fori_loop(0, K // 2, pair, 0)
        wait_chunk(buf_b, (K - 2) * NSUB)
        pltpu.sync_copy(s_neg, out_neg.at[wid])

    return sc_kernel(pu, pv, ng, embed_u, embed_v)


@functools.partial(jax.jit, static_argnames=("B",))
def _tc_loss(pos_score, neg_score, *, B):
    def body(p_ref, n_ref, o_ref):
        def log_sig(x):
            return jnp.minimum(x, 0.0) - jnp.log1p(jnp.exp(-jnp.abs(x)))

        tot = jnp.sum(log_sig(p_ref[...])) + jnp.sum(log_sig(-n_ref[...]))
        o_ref[0, 0] = -tot / B

    out = pl.pallas_call(
        body,
        out_shape=jax.ShapeDtypeStruct((1, 1), jnp.float32),
        out_specs=pl.BlockSpec(memory_space=pltpu.SMEM),
    )(pos_score, neg_score)
    return out[0, 0]


def kernel(pos_u, pos_v, neg_v, embed_u, embed_v):
    B, K = neg_v.shape
    D = embed_u.shape[1]
    pu = pos_u.astype(jnp.int32).reshape(B // 128, 128)
    pv = pos_v.astype(jnp.int32).reshape(B // 128, 128)
    ng = neg_v.astype(jnp.int32).reshape(B * K // 128, 128)
    pos_s, neg_s = _sc_scores(pu, pv, ng, embed_u, embed_v, B=B, K=K, D=D)
    return _tc_loss(pos_s, neg_s, B=B)
